# baseline (speedup 1.0000x reference)
import functools

import jax
import jax.numpy as jnp
from jax import lax
from jax.experimental import pallas as pl
from jax.experimental.pallas import tpu as pltpu
from jax.experimental.pallas import tpu_sc as plsc

B = 16384
D = 32
DW = D // 2
NUM_ITEMS = 1000000
NC = 2
NS = 16
NW = NC * NS
BPW = B // NW
NCHUNK = 4
CHUNK = BPW // NCHUNK
GPC = CHUNK // 16

_TMIN = 1.1920929e-7
_KEXP = 0.1353352832366127
_MAGIC = 0x5F3759DF
_MHI = -65536


def _rsqrt3(x):
    i = plsc.bitcast(x, jnp.int32)
    r = plsc.bitcast(_MAGIC - (i >> 1), jnp.float32)
    r = r * (1.5 - 0.5 * x * r * r)
    r = r * (1.5 - 0.5 * x * r * r)
    r = r * (1.5 - 0.5 * x * r * r)
    return r


def _make_kernel():
    mesh = plsc.VectorSubcoreMesh(core_axis_name="c", subcore_axis_name="s")

    @functools.partial(
        pl.kernel,
        out_type=jax.ShapeDtypeStruct((B,), jnp.float32),
        mesh=mesh,
        compiler_params=pltpu.CompilerParams(
            use_tc_tiling_on_sc=False, needs_layout_passes=False),
        scratch_types=[
            pltpu.VMEM((NCHUNK, CHUNK), jnp.int32),
            pltpu.VMEM((NCHUNK, CHUNK), jnp.int32),
            pltpu.VMEM((BPW, DW), jnp.int32),
            pltpu.VMEM((BPW, DW), jnp.int32),
            pltpu.VMEM((BPW,), jnp.float32),
            pltpu.SemaphoreType.DMA,
            pltpu.SemaphoreType.DMA,
            pltpu.SemaphoreType.DMA,
            pltpu.SemaphoreType.DMA,
        ],
    )
    def lorentz_fd(u_hbm, v_hbm, th_hbm, out_hbm, ui, vi, ru, rv, ov,
                   s0_, s1_, s2_, s3_):
        sems = [s0_, s1_, s2_, s3_]
        wid = lax.axis_index("s") * NC + lax.axis_index("c")
        pltpu.sync_copy(u_hbm.at[wid], ui)
        pltpu.sync_copy(v_hbm.at[wid], vi)
        copies = []
        for j in range(NCHUNK):
            cu = pltpu.async_copy(th_hbm.at[ui.at[j]],
                                  ru.at[pl.ds(j * CHUNK, CHUNK)], sems[j])
            cv = pltpu.async_copy(th_hbm.at[vi.at[j]],
                                  rv.at[pl.ds(j * CHUNK, CHUNK)], sems[j])
            copies.append((cu, cv))

        iota16 = lax.iota(jnp.int32, 16)

        def group_body(g, carry):
            rid = g * 16 + iota16
            a = jnp.zeros((16,), jnp.float32)
            b = jnp.zeros((16,), jnp.float32)
            c = jnp.zeros((16,), jnp.float32)
            for w in range(DW):
                cw = jnp.full((16,), w, jnp.int32)
                wu = plsc.load_gather(ru, [rid, cw])
                wv = plsc.load_gather(rv, [rid, cw])
                hu = plsc.bitcast(wu & _MHI, jnp.float32)
                hv = plsc.bitcast(wv & _MHI, jnp.float32)
                a = a + hu * hu
                b = b + hv * hv
                c = c + hu * hv
                if w > 0:
                    lu = plsc.bitcast(wu << 16, jnp.float32)
                    lv = plsc.bitcast(wv << 16, jnp.float32)
                    a = a + lu * lu
                    b = b + lv * lv
                    c = c + lu * lv
            q = (1.0 + a) * (1.0 + b)
            s0 = q * _rsqrt3(q)
            t0 = (a + b + a * b) / (1.0 + s0)
            t = jnp.maximum(t0 - c, _TMIN)
            w2 = t * (t + 2.0)
            sw = w2 * _rsqrt3(w2)
            ov[pl.ds(g * 16, 16)] = 1.0 / (_KEXP * (1.0 + t + sw) + 1.0)
            return carry

        for j in range(NCHUNK):
            cu, cv = copies[j]
            cu.wait()
            cv.wait()
            lax.fori_loop(j * GPC, (j + 1) * GPC, group_body, 0)

        pltpu.sync_copy(ov, out_hbm.at[pl.ds(wid * BPW, BPW)])

    return lorentz_fd


_lorentz = _make_kernel()


def kernel(u, v, theta):
    u3 = u.astype(jnp.int32).reshape(NW, NCHUNK, CHUNK)
    v3 = v.astype(jnp.int32).reshape(NW, NCHUNK, CHUNK)
    th_bf = theta.astype(jnp.bfloat16)
    th_i = jax.lax.bitcast_convert_type(
        th_bf.reshape(NUM_ITEMS, DW, 2), jnp.int32)
    return _lorentz(u3, v3, th_i)
